# baseline (speedup 1.0000x reference)
import functools

import jax
import jax.numpy as jnp
from jax import lax
from jax.experimental import pallas as pl
from jax.experimental.pallas import tpu as pltpu
from jax.experimental.pallas import tpu_sc as plsc

_INV_SQRT_2PI = 0.3989422804014327
_INV_PI = 0.3183098861837907
_PI_HI = 3.140625
_PI_MID = 0.0009676535897932384
_C3 = -1.0 / 6.0
_C5 = 1.0 / 120.0
_C7 = -1.0 / 5040.0
_C9 = 1.0 / 362880.0

_NW = 32
_CHUNK = 2048
_CB = _CHUNK // 128


def _bfv(v):
    b = lax.bitcast_convert_type(v, jnp.int32)
    r = (b + 32767 + ((b >> 16) & 1)) & (-65536)
    return lax.bitcast_convert_type(r, jnp.float32)


def _sin(s):
    t = s * _INV_PI
    half = jnp.where(t >= 0.0, 0.5, -0.5)
    ni = (t + half).astype(jnp.int32)
    nf = ni.astype(jnp.float32)
    sign = jnp.where((ni & 1) == 1, -1.0, 1.0)
    r = (s - nf * _PI_HI) - nf * _PI_MID
    r2 = r * r
    p = r * (1.0 + r2 * (_C3 + r2 * (_C5 + r2 * (_C7 + r2 * _C9))))
    return sign * p


def _make(N):
    M = N // 128
    rows_w = N // _NW
    n_chunks = rows_w // _CHUNK
    mesh = plsc.VectorSubcoreMesh(core_axis_name="c", subcore_axis_name="s")

    @functools.partial(
        pl.kernel,
        mesh=mesh,
        out_type=jax.ShapeDtypeStruct((4 * M * 128,), jnp.float32),
        scratch_types=[
            pltpu.VMEM((64,), jnp.float32),
            pltpu.VMEM((64,), jnp.float32),
            pltpu.VMEM((_CHUNK,), jnp.float32),
            pltpu.VMEM((_CHUNK,), jnp.float32),
            pltpu.VMEM((2 * _CB * 1024,), jnp.float32),
            pltpu.VMEM((4 * _CHUNK,), jnp.float32),
        ],
    )
    def sck(wf_hbm, wb_hbm, x_hbm, y_hbm, z_hbm, out_hbm,
            wf_v, wb_v, x_v, y_v, z_v, o_v):
        wid = lax.axis_index("s") * 2 + lax.axis_index("c")
        pltpu.sync_copy(wf_hbm, wf_v)
        pltpu.sync_copy(wb_hbm, wb_v)

        wa = wf_v[pl.ds(0, 16)]
        wz1 = wb_v[pl.ds(16, 16)]
        wz2 = wb_v[pl.ds(32, 16)]
        wo = wb_v[pl.ds(48, 16)]

        row0_w = wid * rows_w
        for c in range(n_chunks):
            row0 = row0_w + c * _CHUNK
            cb0 = row0 // 128
            pltpu.sync_copy(x_hbm.at[pl.ds(row0, _CHUNK)], x_v)
            pltpu.sync_copy(y_hbm.at[pl.ds(row0, _CHUNK)], y_v)
            for rb in range(2):
                pltpu.sync_copy(
                    z_hbm.at[pl.ds(rb * M * 1024 + cb0 * 1024, _CB * 1024)],
                    z_v.at[pl.ds(rb * _CB * 1024, _CB * 1024)])

            def body(t, carry):
                cbl = t // 8
                l0 = (t % 8) * 16
                x = x_v[pl.ds(cbl * 128 + l0, 16)]
                y = y_v[pl.ds(cbl * 128 + l0, 16)]
                s1 = wa[0] * x + wa[1] * y
                s2 = wa[2] * x + wa[3] * y
                for kp in range(8):
                    k0, k1 = 2 * kp, 2 * kp + 1
                    rb0, sb0 = k0 // 8, k0 % 8
                    rb1, sb1 = k1 // 8, k1 % 8
                    za = z_v[pl.ds(((rb0 * _CB + cbl) * 8 + sb0) * 128 + l0, 16)]
                    zc = z_v[pl.ds(((rb1 * _CB + cbl) * 8 + sb1) * 128 + l0, 16)]
                    zra, zrb = _bfv(za), _bfv(zc)
                    s1 = s1 + wz1[k0] * zra + wz1[k1] * zrb
                    s2 = s2 + wz2[k0] * zra + wz2[k1] * zrb
                h1 = _sin(s1)
                pre2 = s2 + wa[4] * h1
                h2 = _INV_SQRT_2PI * jnp.exp(-0.5 * pre2 * pre2)
                h1b, h2b = _bfv(h1), _bfv(h2)
                for j in range(3):
                    p = wo[j] * h1b + wo[3 + j] * h2b
                    o_v[pl.ds((4 * cbl + j) * 128 + l0, 16)] = 1.0 / (1.0 + jnp.exp(-p))
                o_v[pl.ds((4 * cbl + 3) * 128 + l0, 16)] = jnp.zeros((16,), jnp.float32)
                return carry

            lax.fori_loop(0, _CB * 8, body, 0)
            pltpu.sync_copy(o_v, out_hbm.at[pl.ds(4 * row0, 4 * _CHUNK)])

    return sck


@jax.jit
def _run_sc(X, Y, Z, w1, w2, w_out):
    N = X.shape[0]
    M = N // 128
    Xf = X.reshape(N)
    Yf = Y.reshape(N)
    Zf = Z.reshape(M, 128, 2, 8).transpose(2, 0, 3, 1).reshape(2 * M * 8 * 128)

    z5 = jnp.zeros((11,), jnp.float32)
    z10 = jnp.zeros((10,), jnp.float32)
    wf = jnp.concatenate([w1[:2], w2[:3], z5, w1[2:], w2[3:],
                          w_out[0], w_out[1], z10])
    wb = wf.astype(jnp.bfloat16).astype(jnp.float32)

    out_flat = _make(N)(wf, wb, Xf, Yf, Zf)
    o4 = out_flat.reshape(4 * M, 128)
    return o4.reshape(M, 4, 128).transpose(0, 2, 1).reshape(N, 4)[:, :3]


def kernel(X, Y, R, Z, w1, w2, w_out):
    del R
    return _run_sc(X, Y, Z, w1, w2, w_out)


# speedup vs baseline: 4.9381x; 4.9381x over previous
import jax
import jax.numpy as jnp
from jax.experimental import pallas as pl
from jax.experimental.pallas import tpu as pltpu

_INV_SQRT_2PI = 0.3989422804014327


def _bf(v):
    return v.astype(jnp.bfloat16).astype(jnp.float32)


def _row(w_ref, i):
    return jnp.full((1, 128), w_ref[i], dtype=jnp.float32)


def _cppn_body(w1_ref, w2_ref, wo_ref, x_ref, y_ref, z_ref, out_ref, wz_ref):
    @pl.when(pl.program_id(0) == 0)
    def _init():
        wz_ref[0] = _bf(jnp.concatenate([_row(w1_ref, 2 + s) for s in range(8)], 0))
        wz_ref[1] = _bf(jnp.concatenate([_row(w1_ref, 10 + s) for s in range(8)], 0))
        wz_ref[2] = _bf(jnp.concatenate([_row(w2_ref, 3 + s) for s in range(8)], 0))
        wz_ref[3] = _bf(jnp.concatenate([_row(w2_ref, 11 + s) for s in range(8)], 0))

    x = x_ref[...]
    y = y_ref[...]
    zb0 = _bf(z_ref[0])
    zb1 = _bf(z_ref[1])
    s1 = (w1_ref[0] * x + w1_ref[1] * y
          + jnp.sum(zb0 * wz_ref[0] + zb1 * wz_ref[1], axis=1))
    s2 = (w2_ref[0] * x + w2_ref[1] * y
          + jnp.sum(zb0 * wz_ref[2] + zb1 * wz_ref[3], axis=1))
    h1 = jnp.sin(s1)
    pre2 = s2 + w2_ref[2] * h1
    h2 = _INV_SQRT_2PI * jnp.exp(-0.5 * pre2 * pre2)
    h1b = _bf(h1)
    h2b = _bf(h2)
    o = []
    for j in range(3):
        p = _bf(wo_ref[0, j]) * h1b + _bf(wo_ref[1, j]) * h2b
        o.append(1.0 / (1.0 + jnp.exp(-p)))
    o.append(o[2])
    out_ref[...] = jnp.stack(o, axis=1).reshape(out_ref.shape)


@jax.jit
def _run(X, Y, Z, w1, w2, w_out):
    N = X.shape[0]
    M = N // 128
    BM = 256

    Xr = X.reshape(M, 128)
    Yr = Y.reshape(M, 128)
    Zr = Z.reshape(M, 128, 2, 8).transpose(2, 0, 3, 1)

    out4 = pl.pallas_call(
        _cppn_body,
        grid=(M // BM,),
        in_specs=[
            pl.BlockSpec(memory_space=pltpu.SMEM),
            pl.BlockSpec(memory_space=pltpu.SMEM),
            pl.BlockSpec(memory_space=pltpu.SMEM),
            pl.BlockSpec((BM, 128), lambda i: (i, 0)),
            pl.BlockSpec((BM, 128), lambda i: (i, 0)),
            pl.BlockSpec((2, BM, 8, 128), lambda i: (0, i, 0, 0)),
        ],
        out_specs=pl.BlockSpec((4 * BM, 128), lambda i: (i, 0)),
        out_shape=jax.ShapeDtypeStruct((4 * M, 128), jnp.float32),
        scratch_shapes=[pltpu.VMEM((4, 8, 128), jnp.float32)],
    )(w1, w2, w_out, Xr, Yr, Zr)
    return out4.reshape(M, 4, 128).transpose(0, 2, 1).reshape(N, 4)[:, :3]


def kernel(X, Y, R, Z, w1, w2, w_out):
    del R
    return _run(X, Y, Z, w1, w2, w_out)
